# baseline (speedup 1.0000x reference)
import functools

import jax
import jax.numpy as jnp
from jax.experimental import pallas as pl
from jax.experimental.pallas import tpu as pltpu

N0 = 10000
N2 = 10000
NEG_SLOPE = 0.2

_BLK = 1000


def _mm_body(x_ref, w_ref, c_ref, y_ref, pq_ref):
    y = jnp.dot(x_ref[...], w_ref[...], preferred_element_type=jnp.float32)
    y_ref[...] = y
    pq_ref[...] = jnp.dot(y, c_ref[...], preferred_element_type=jnp.float32)


@jax.jit
def _mm_proj(x, w, c_pad):
    n, d_in = x.shape
    d_out = w.shape[1]
    grid = (n // _BLK,)
    return pl.pallas_call(
        _mm_body,
        grid=grid,
        in_specs=[
            pl.BlockSpec((_BLK, d_in), lambda i: (i, 0)),
            pl.BlockSpec((d_in, d_out), lambda i: (0, 0)),
            pl.BlockSpec((d_out, 128), lambda i: (0, 0)),
        ],
        out_specs=[
            pl.BlockSpec((_BLK, d_out), lambda i: (i, 0)),
            pl.BlockSpec((_BLK, 128), lambda i: (i, 0)),
        ],
        out_shape=[
            jax.ShapeDtypeStruct((n, d_out), jnp.float32),
            jax.ShapeDtypeStruct((n, 128), jnp.float32),
        ],
    )(x, w, c_pad)


def _leaky(v):
    return jnp.where(v >= 0, v, NEG_SLOPE * v)


def _soft_edge(pr, pc, rid, cid, n_rows):
    ev = jnp.exp(_leaky(pr[rid] + pc[cid]))
    s = jax.ops.segment_sum(ev, rid, num_segments=n_rows)
    return ev / jnp.maximum(s[rid], 1e-30)


def kernel(x_0, x_2, adjacency_0, incidence_0_2, w_hbs, att_hbs, w_s, w_t, att_hbns):
    def cpad(c0, c1):
        z = jnp.zeros((256, 128), jnp.float32)
        z = z.at[:, 0].set(c0)
        return z.at[:, 1].set(c1)

    c_hbs = cpad(att_hbs[:256, 0], att_hbs[256:, 0])
    c_t = cpad(att_hbns[256:, 0], att_hbns[:256, 0])
    c_s = cpad(att_hbns[:256, 0], att_hbns[256:, 0])

    msg, pq = _mm_proj(x_0, w_hbs, c_hbs)
    t_msg, atbt = _mm_proj(x_0, w_t, c_t)
    s_msg, asbs = _mm_proj(x_2, w_s, c_s)

    p, q = pq[:, 0], pq[:, 1]
    at, bt = atbt[:, 0], atbt[:, 1]
    as_, bs = asbs[:, 0], asbs[:, 1]

    ai, aj = adjacency_0[0], adjacency_0[1]
    ti, sj = incidence_0_2[0], incidence_0_2[1]

    att = _soft_edge(p, q, ai, aj, N0)
    x_0_to_0 = jax.ops.segment_sum(att[:, None] * msg[aj], ai, num_segments=N0)

    e2 = _soft_edge(at, as_, ti, sj, N0)
    x_2_to_0 = jax.ops.segment_sum(e2[:, None] * s_msg[sj], ti, num_segments=N0)

    f2 = _soft_edge(bs, bt, sj, ti, N2)
    x_0_to_2 = jax.ops.segment_sum(f2[:, None] * t_msg[ti], sj, num_segments=N2)

    return (x_0_to_0 + x_2_to_0, x_0_to_2)


# speedup vs baseline: 5.6740x; 5.6740x over previous
import functools

import jax
import jax.numpy as jnp
from jax import lax
from jax.experimental import pallas as pl
from jax.experimental.pallas import tpu as pltpu
from jax.experimental.pallas import tpu_sc as plsc

N0 = 10000
N2 = 10000
E = 160000
NNZ = 200000
NEG_SLOPE = 0.2

NP = 10240
NPR = NP // 128
NPASS = 2
NH = NP // NPASS
NHA = NH + 32
NNZP = 200192
ET_A = E // 16
ET_I = NNZP // 16
RPT = NH // 16
CAP = 1024
SCB = 62

_BLK = 1000

_mesh = plsc.VectorSubcoreMesh(core_axis_name="c", subcore_axis_name="s",
                               num_cores=2, num_subcores=16)
_CP = pltpu.CompilerParams(needs_layout_passes=False)


def _mm_body(x_ref, w_ref, c_ref, y_ref, pq_ref):
    y = jnp.dot(x_ref[...], w_ref[...], preferred_element_type=jnp.float32)
    y_ref[...] = y
    pq_ref[...] = jnp.dot(y, c_ref[...], preferred_element_type=jnp.float32)


def _mm_proj(x, w, c_pad):
    n, d_in = x.shape
    d_out = w.shape[1]
    return pl.pallas_call(
        _mm_body,
        grid=(n // _BLK,),
        in_specs=[
            pl.BlockSpec((_BLK, d_in), lambda i: (i, 0)),
            pl.BlockSpec((d_in, d_out), lambda i: (0, 0)),
            pl.BlockSpec((d_out, 128), lambda i: (0, 0)),
        ],
        out_specs=[
            pl.BlockSpec((_BLK, d_out), lambda i: (i, 0)),
            pl.BlockSpec((_BLK, 128), lambda i: (i, 0)),
        ],
        out_shape=[
            jax.ShapeDtypeStruct((n, d_out), jnp.float32),
            jax.ShapeDtypeStruct((n, 128), jnp.float32),
        ],
    )(x, w, c_pad)


@functools.partial(
    pl.kernel,
    out_type=[jax.ShapeDtypeStruct((2, NPASS, NH, 128), jnp.float32),
              jax.ShapeDtypeStruct((2, NPASS, NH, 128), jnp.float32)],
    mesh=_mesh,
    scratch_types=[
        pltpu.VMEM((ET_I,), jnp.int32),
        pltpu.VMEM((ET_I,), jnp.int32),
        pltpu.VMEM((ET_I,), jnp.float32),
        pltpu.VMEM((NP,), jnp.float32),
        pltpu.VMEM((NP,), jnp.float32),
        pltpu.VMEM((NPR, 128), jnp.float32),
        pltpu.VMEM((CAP,), jnp.int32),
        pltpu.VMEM((CAP,), jnp.int32),
        pltpu.VMEM((CAP,), jnp.float32),
        pltpu.VMEM((16, 128), jnp.float32),
        pltpu.VMEM((16, 128), jnp.float32),
        pltpu.VMEM((16, 128), jnp.float32),
        pltpu.VMEM((16, 128), jnp.float32),
        pltpu.VMEM((16,), jnp.int32),
        pltpu.VMEM((16,), jnp.int32),
        pltpu.VMEM((NPR,), jnp.int32),
        pltpu.VMEM_SHARED((NPR, 128), jnp.float32),
        pltpu.VMEM_SHARED((NHA, 128), jnp.float32),
        pltpu.SemaphoreType.DMA,
    ],
    compiler_params=_CP,
)
def _sparse_sc(msgT, smsgT, tmsgT, scal, ai_h, aj_h, ti_h, sj_h,
               out0, out2,
               r_v, c_v, att_v, pr_v, pc_v, sp_v, srid, scid, satt,
               rowbuf, zbuf, stg, stg2, ridb, gidb, rowids, s_sh, acc, sem):
    c = lax.axis_index("c")
    t = lax.axis_index("s")
    zero16 = jnp.zeros((16,), jnp.float32)
    iota16 = lax.iota(jnp.int32, 16)
    nsr = NPR // 16

    def _zb(i, _):
        for v in range(8):
            zbuf[i, pl.ds(16 * v, 16)] = zero16
        return 0
    lax.fori_loop(0, 16, _zb, 0)

    def _fri(k, _):
        rowids[pl.ds(16 * k, 16)] = iota16 + 16 * k
        return 0
    lax.fori_loop(0, NPR // 16, _fri, 0)

    def _za(j, _):
        pltpu.sync_copy(zbuf, acc.at[pl.ds(t * RPT + 16 * j, 16)])
        return 0
    lax.fori_loop(0, RPT // 16, _za, 0)

    @pl.when(t == 0)
    def _():
        pltpu.sync_copy(zbuf, acc.at[pl.ds(NH, 16)])
        pltpu.sync_copy(zbuf, acc.at[pl.ds(NH + 16, 16)])

    pltpu.sync_copy(zbuf.at[pl.ds(0, nsr)], s_sh.at[pl.ds(nsr * t, nsr)])
    plsc.subcore_barrier()

    def scalar_stage(et, pr_row, pc_row):
        nch = et // 16
        pltpu.sync_copy(scal.at[pr_row], pr_v)
        pltpu.sync_copy(scal.at[pc_row], pc_v)

        def _zs(i, _):
            for v in range(8):
                sp_v[i, pl.ds(16 * v, 16)] = zero16
            return 0
        lax.fori_loop(0, NPR, _zs, 0)

        def _sta(k, _):
            rid = r_v[pl.ds(16 * k, 16)]
            cid = c_v[pl.ds(16 * k, 16)]
            e = (plsc.load_gather(pr_v, [rid])
                 + plsc.load_gather(pc_v, [cid]))
            e = jnp.where(e >= 0, e, NEG_SLOPE * e)
            ev = jnp.exp(e)
            att_v[pl.ds(16 * k, 16)] = ev
            plsc.addupdate_scatter(
                sp_v, [jnp.right_shift(rid, 7), jnp.bitwise_and(rid, 127)], ev)
            return 0
        lax.fori_loop(0, nch, _sta, 0)

        pltpu.sync_copy(sp_v, s_sh.at[rowids], add=True)
        plsc.subcore_barrier()
        pltpu.sync_copy(s_sh, sp_v)

        def _stb(k, _):
            rid = r_v[pl.ds(16 * k, 16)]
            sv = plsc.load_gather(
                sp_v, [jnp.right_shift(rid, 7), jnp.bitwise_and(rid, 127)])
            att = att_v[pl.ds(16 * k, 16)] / jnp.maximum(sv, 1e-30)
            att_v[pl.ds(16 * k, 16)] = att
            return 0
        lax.fori_loop(0, nch, _stb, 0)

        pltpu.sync_copy(zbuf.at[pl.ds(0, nsr)], s_sh.at[pl.ds(nsr * t, nsr)])
        plsc.subcore_barrier()

    def _drain_chunk(tab, k, _):
        rid = srid[pl.ds(16 * k, 16)]
        cid = scid[pl.ds(16 * k, 16)]
        ridb[...] = rid
        gidb[...] = cid * 2 + c
        pltpu.async_copy(tab.at[gidb], rowbuf, sem).wait()
        att_vec = satt[pl.ds(16 * k, 16)]
        for j in range(16):
            av = jnp.full((16,), att_vec[j], jnp.float32)
            for v in range(8):
                sl = pl.ds(16 * v, 16)
                rowbuf[j, sl] = rowbuf[j, sl] * av
        pltpu.sync_copy(rowbuf, acc.at[ridb], add=True)
        return 0

    def heavy(tab, et, p):
        nch = et // 16
        lo = p * NH

        def _scan(k, cur):
            rid = r_v[pl.ds(16 * k, 16)] - lo
            cid = c_v[pl.ds(16 * k, 16)]
            att = att_v[pl.ds(16 * k, 16)]
            mask = jnp.logical_and(rid >= 0, rid < NH)
            plsc.store_compressed(srid.at[pl.ds(cur, 16)], rid, mask=mask)
            plsc.store_compressed(scid.at[pl.ds(cur, 16)], cid, mask=mask)
            plsc.store_compressed(satt.at[pl.ds(cur, 16)], att, mask=mask)
            return cur + plsc.all_reduce_population_count(mask)[0]

        def _drain(cur):
            full = cur // 16
            lax.fori_loop(0, full, functools.partial(_drain_chunk, tab), 0)
            rv = srid[pl.ds(16 * full, 16)]
            cv = scid[pl.ds(16 * full, 16)]
            av = satt[pl.ds(16 * full, 16)]
            srid[pl.ds(0, 16)] = rv
            scid[pl.ds(0, 16)] = cv
            satt[pl.ds(0, 16)] = av
            return cur - full * 16

        nblk = (nch + SCB - 1) // SCB

        def _blk(b, cur):
            start = b * SCB
            end = jnp.minimum(start + SCB, nch)
            cur = lax.fori_loop(start, end, _scan, cur)
            return _drain(cur)

        cur = lax.fori_loop(0, nblk, _blk, jnp.int32(0))
        rv = srid[pl.ds(0, 16)]
        cv = scid[pl.ds(0, 16)]
        av = satt[pl.ds(0, 16)]
        live = iota16 < cur
        srid[pl.ds(0, 16)] = jnp.where(live, rv, NH)
        scid[pl.ds(0, 16)] = jnp.where(live, cv, 0)
        satt[pl.ds(0, 16)] = jnp.where(live, av, 0.0)
        _drain_chunk(tab, 0, 0)

    def writeback_zero(out, p, add_prev):
        def _wb(j, _):
            r0 = t * RPT + 16 * j
            pltpu.sync_copy(acc.at[pl.ds(r0, 16)], stg)
            if add_prev:
                pltpu.sync_copy(out.at[c, p, pl.ds(r0, 16)], stg2)

                def _addrow(i, _):
                    for v in range(8):
                        sl = pl.ds(16 * v, 16)
                        stg[i, sl] = stg[i, sl] + stg2[i, sl]
                    return 0
                lax.fori_loop(0, 16, _addrow, 0)
            pltpu.sync_copy(stg, out.at[c, p, pl.ds(r0, 16)])
            pltpu.sync_copy(zbuf, acc.at[pl.ds(r0, 16)])
            return 0
        lax.fori_loop(0, RPT // 16, _wb, 0)

    def round_(tab, rhbm, chbm, et, pr_row, pc_row, out, add_prev):
        pltpu.sync_copy(rhbm.at[pl.ds(t * et, et)], r_v.at[pl.ds(0, et)])
        pltpu.sync_copy(chbm.at[pl.ds(t * et, et)], c_v.at[pl.ds(0, et)])
        scalar_stage(et, pr_row, pc_row)

        def _pass(p, _):
            heavy(tab, et, p)
            plsc.subcore_barrier()
            writeback_zero(out, p, add_prev)
            plsc.subcore_barrier()
            return 0
        lax.fori_loop(0, NPASS, _pass, 0)

    round_(msgT, ai_h, aj_h, ET_A, 0, 1, out0, False)
    round_(smsgT, ti_h, sj_h, ET_I, 2, 3, out0, True)
    round_(tmsgT, sj_h, ti_h, ET_I, 5, 4, out2, False)


def kernel(x_0, x_2, adjacency_0, incidence_0_2, w_hbs, att_hbs, w_s, w_t, att_hbns):
    def cpad(c0, c1):
        z = jnp.zeros((256, 128), jnp.float32)
        z = z.at[:, 0].set(c0)
        return z.at[:, 1].set(c1)

    c_hbs = cpad(att_hbs[:256, 0], att_hbs[256:, 0])
    c_t = cpad(att_hbns[256:, 0], att_hbns[:256, 0])
    c_s = cpad(att_hbns[:256, 0], att_hbns[256:, 0])

    msg, pq = _mm_proj(x_0, w_hbs, c_hbs)
    t_msg, atbt = _mm_proj(x_0, w_t, c_t)
    s_msg, asbs = _mm_proj(x_2, w_s, c_s)

    npad = NP - N0

    def tabify(y):
        return jnp.pad(y, ((0, npad), (0, 0))).reshape(2 * NP, 128)

    def svec(v):
        return jnp.pad(v, (0, npad))

    scal = jnp.stack([svec(pq[:, 0]), svec(pq[:, 1]),
                      svec(atbt[:, 0]), svec(asbs[:, 0]),
                      svec(atbt[:, 1]), svec(asbs[:, 1])])

    ipad = NNZP - NNZ
    ti = jnp.pad(incidence_0_2[0], (0, ipad), constant_values=N0)
    sj = jnp.pad(incidence_0_2[1], (0, ipad), constant_values=N0)

    out0, out2 = _sparse_sc(tabify(msg), tabify(s_msg), tabify(t_msg), scal,
                            adjacency_0[0], adjacency_0[1], ti, sj)

    def assemble(o, n):
        cols = [o[cc].reshape(NPASS * NH, 128) for cc in (0, 1)]
        return jnp.concatenate(cols, axis=1)[:n]

    return (assemble(out0, N0), assemble(out2, N2))


# speedup vs baseline: 9.6786x; 1.7058x over previous
import functools

import jax
import jax.numpy as jnp
from jax import lax
from jax.experimental import pallas as pl
from jax.experimental.pallas import tpu as pltpu
from jax.experimental.pallas import tpu_sc as plsc

N0 = 10000
N2 = 10000
E = 160000
NNZ = 200000
NEG_SLOPE = 0.2

NP = 10240
NPR = NP // 128
NPASS = 2
NH = NP // NPASS
NHA = NH + 32
NNZP = 200192
ET_A = E // 16
ET_I = NNZP // 16
RPT = NH // 16
CAP = 1024
SCB = 62

_BLK = 1000

_mesh = plsc.VectorSubcoreMesh(core_axis_name="c", subcore_axis_name="s",
                               num_cores=2, num_subcores=16)
_CP = pltpu.CompilerParams(needs_layout_passes=False)


def _mm_body(x_ref, w_ref, c_ref, y_ref, pq_ref):
    y = jnp.dot(x_ref[...], w_ref[...], preferred_element_type=jnp.float32)
    y_ref[...] = y
    pq_ref[...] = jnp.dot(y, c_ref[...], preferred_element_type=jnp.float32)


def _mm_proj(x, w, c_pad):
    n, d_in = x.shape
    d_out = w.shape[1]
    return pl.pallas_call(
        _mm_body,
        grid=(n // _BLK,),
        in_specs=[
            pl.BlockSpec((_BLK, d_in), lambda i: (i, 0)),
            pl.BlockSpec((d_in, d_out), lambda i: (0, 0)),
            pl.BlockSpec((d_out, 128), lambda i: (0, 0)),
        ],
        out_specs=[
            pl.BlockSpec((_BLK, d_out), lambda i: (i, 0)),
            pl.BlockSpec((_BLK, 128), lambda i: (i, 0)),
        ],
        out_shape=[
            jax.ShapeDtypeStruct((n, d_out), jnp.float32),
            jax.ShapeDtypeStruct((n, 128), jnp.float32),
        ],
    )(x, w, c_pad)


@functools.partial(
    pl.kernel,
    out_type=[jax.ShapeDtypeStruct((2, NPASS, NH, 128), jnp.float32),
              jax.ShapeDtypeStruct((2, NPASS, NH, 128), jnp.float32)],
    mesh=_mesh,
    scratch_types=[
        pltpu.VMEM((ET_I,), jnp.int32),
        pltpu.VMEM((ET_I,), jnp.int32),
        pltpu.VMEM((ET_I,), jnp.float32),
        pltpu.VMEM((NP,), jnp.float32),
        pltpu.VMEM((NP,), jnp.float32),
        pltpu.VMEM((NPR, 128), jnp.float32),
        pltpu.VMEM((CAP,), jnp.int32),
        pltpu.VMEM((CAP,), jnp.int32),
        pltpu.VMEM((CAP,), jnp.float32),
        pltpu.VMEM((16, 128), jnp.float32),
        pltpu.VMEM((16, 128), jnp.float32),
        pltpu.VMEM((16, 128), jnp.float32),
        pltpu.VMEM((16, 128), jnp.float32),
        pltpu.VMEM((16, 128), jnp.float32),
        pltpu.VMEM((16, 128), jnp.float32),
        pltpu.VMEM((16, 128), jnp.float32),
        pltpu.VMEM((16,), jnp.int32),
        pltpu.VMEM((16,), jnp.int32),
        pltpu.VMEM((16,), jnp.int32),
        pltpu.VMEM((16,), jnp.int32),
        pltpu.VMEM((16,), jnp.int32),
        pltpu.VMEM((16,), jnp.int32),
        pltpu.VMEM((16,), jnp.int32),
        pltpu.VMEM((16,), jnp.int32),
        pltpu.VMEM((NPR,), jnp.int32),
        pltpu.VMEM_SHARED((NPR, 128), jnp.float32),
        pltpu.VMEM_SHARED((NHA, 128), jnp.float32),
        pltpu.SemaphoreType.DMA,
    ],
    compiler_params=_CP,
)
def _sparse_sc(msgT, smsgT, tmsgT, scal, ai_h, aj_h, ti_h, sj_h,
               out0, out2,
               r_v, c_v, att_v, pr_v, pc_v, sp_v, srid, scid, satt,
               rowbuf, rowbuf1, rowbuf2, rowbuf3, zbuf, stg, stg2,
               ridb, ridb1, ridb2, ridb3, gidb, gidb1, gidb2, gidb3,
               rowids, s_sh, acc, sem):
    rowbufs = (rowbuf, rowbuf1, rowbuf2, rowbuf3)
    ridbs = (ridb, ridb1, ridb2, ridb3)
    gidbs = (gidb, gidb1, gidb2, gidb3)
    c = lax.axis_index("c")
    t = lax.axis_index("s")
    zero16 = jnp.zeros((16,), jnp.float32)
    iota16 = lax.iota(jnp.int32, 16)
    nsr = NPR // 16

    def _zb(i, _):
        for v in range(8):
            zbuf[i, pl.ds(16 * v, 16)] = zero16
        return 0
    lax.fori_loop(0, 16, _zb, 0)

    def _fri(k, _):
        rowids[pl.ds(16 * k, 16)] = iota16 + 16 * k
        return 0
    lax.fori_loop(0, NPR // 16, _fri, 0)

    def _za(j, _):
        pltpu.sync_copy(zbuf, acc.at[pl.ds(t * RPT + 16 * j, 16)])
        return 0
    lax.fori_loop(0, RPT // 16, _za, 0)

    @pl.when(t == 0)
    def _():
        pltpu.sync_copy(zbuf, acc.at[pl.ds(NH, 16)])
        pltpu.sync_copy(zbuf, acc.at[pl.ds(NH + 16, 16)])

    pltpu.sync_copy(zbuf.at[pl.ds(0, nsr)], s_sh.at[pl.ds(nsr * t, nsr)])
    plsc.subcore_barrier()

    def scalar_stage(et, pr_row, pc_row):
        nch = et // 16
        pltpu.sync_copy(scal.at[pr_row], pr_v)
        pltpu.sync_copy(scal.at[pc_row], pc_v)

        def _zs(i, _):
            for v in range(8):
                sp_v[i, pl.ds(16 * v, 16)] = zero16
            return 0
        lax.fori_loop(0, NPR, _zs, 0)

        def _sta(k, _):
            rid = r_v[pl.ds(16 * k, 16)]
            cid = c_v[pl.ds(16 * k, 16)]
            e = (plsc.load_gather(pr_v, [rid])
                 + plsc.load_gather(pc_v, [cid]))
            e = jnp.where(e >= 0, e, NEG_SLOPE * e)
            ev = jnp.exp(e)
            att_v[pl.ds(16 * k, 16)] = ev
            plsc.addupdate_scatter(
                sp_v, [jnp.right_shift(rid, 7), jnp.bitwise_and(rid, 127)], ev)
            return 0
        lax.fori_loop(0, nch, _sta, 0)

        pltpu.sync_copy(sp_v, s_sh.at[rowids], add=True)
        plsc.subcore_barrier()
        pltpu.sync_copy(s_sh, sp_v)

        def _stb(k, _):
            rid = r_v[pl.ds(16 * k, 16)]
            sv = plsc.load_gather(
                sp_v, [jnp.right_shift(rid, 7), jnp.bitwise_and(rid, 127)])
            att = att_v[pl.ds(16 * k, 16)] / jnp.maximum(sv, 1e-30)
            att_v[pl.ds(16 * k, 16)] = att
            return 0
        lax.fori_loop(0, nch, _stb, 0)

        pltpu.sync_copy(zbuf.at[pl.ds(0, nsr)], s_sh.at[pl.ds(nsr * t, nsr)])
        plsc.subcore_barrier()

    def _drain_chunk(tab, k, _):
        rid = srid[pl.ds(16 * k, 16)]
        cid = scid[pl.ds(16 * k, 16)]
        ridb[...] = rid
        gidb[...] = cid * 2 + c
        pltpu.async_copy(tab.at[gidb], rowbuf, sem).wait()
        att_vec = satt[pl.ds(16 * k, 16)]
        for j in range(16):
            av = jnp.full((16,), att_vec[j], jnp.float32)
            for v in range(8):
                sl = pl.ds(16 * v, 16)
                rowbuf[j, sl] = rowbuf[j, sl] * av
        pltpu.sync_copy(rowbuf, acc.at[ridb], add=True)
        return 0

    def heavy(tab, et, p):
        nch = et // 16
        lo = p * NH

        def _scan(k, cur):
            rid = r_v[pl.ds(16 * k, 16)] - lo
            cid = c_v[pl.ds(16 * k, 16)]
            att = att_v[pl.ds(16 * k, 16)]
            mask = jnp.logical_and(rid >= 0, rid < NH)
            plsc.store_compressed(srid.at[pl.ds(cur, 16)], rid, mask=mask)
            plsc.store_compressed(scid.at[pl.ds(cur, 16)], cid, mask=mask)
            plsc.store_compressed(satt.at[pl.ds(cur, 16)], att, mask=mask)
            return cur + plsc.all_reduce_population_count(mask)[0]

        def _grp(b, _):
            base = 4 * b
            for s in range(4):
                k = base + s
                ridbs[s][...] = srid[pl.ds(16 * k, 16)]
                gidbs[s][...] = scid[pl.ds(16 * k, 16)] * 2 + c
            descs = [pltpu.async_copy(tab.at[gidbs[s]], rowbufs[s], sem)
                     for s in range(4)]
            for s in range(4):
                k = base + s
                descs[s].wait()
                att_vec = satt[pl.ds(16 * k, 16)]
                for j in range(16):
                    av = jnp.full((16,), att_vec[j], jnp.float32)
                    for v in range(8):
                        sl = pl.ds(16 * v, 16)
                        rowbufs[s][j, sl] = rowbufs[s][j, sl] * av
                pltpu.sync_copy(rowbufs[s], acc.at[ridbs[s]], add=True)
            return 0

        def _drain(cur):
            full = cur // 16
            ngrp = full // 4
            lax.fori_loop(0, ngrp, _grp, 0)
            lax.fori_loop(4 * ngrp, full, functools.partial(_drain_chunk, tab), 0)
            rv = srid[pl.ds(16 * full, 16)]
            cv = scid[pl.ds(16 * full, 16)]
            av = satt[pl.ds(16 * full, 16)]
            srid[pl.ds(0, 16)] = rv
            scid[pl.ds(0, 16)] = cv
            satt[pl.ds(0, 16)] = av
            return cur - full * 16

        nblk = (nch + SCB - 1) // SCB

        def _blk(b, cur):
            start = b * SCB
            end = jnp.minimum(start + SCB, nch)
            cur = lax.fori_loop(start, end, _scan, cur)
            return _drain(cur)

        cur = lax.fori_loop(0, nblk, _blk, jnp.int32(0))
        rv = srid[pl.ds(0, 16)]
        cv = scid[pl.ds(0, 16)]
        av = satt[pl.ds(0, 16)]
        live = iota16 < cur
        srid[pl.ds(0, 16)] = jnp.where(live, rv, NH)
        scid[pl.ds(0, 16)] = jnp.where(live, cv, 0)
        satt[pl.ds(0, 16)] = jnp.where(live, av, 0.0)
        _drain_chunk(tab, 0, 0)

    def writeback_zero(out, p, add_prev):
        def _wb(j, _):
            r0 = t * RPT + 16 * j
            pltpu.sync_copy(acc.at[pl.ds(r0, 16)], stg)
            if add_prev:
                pltpu.sync_copy(out.at[c, p, pl.ds(r0, 16)], stg2)

                def _addrow(i, _):
                    for v in range(8):
                        sl = pl.ds(16 * v, 16)
                        stg[i, sl] = stg[i, sl] + stg2[i, sl]
                    return 0
                lax.fori_loop(0, 16, _addrow, 0)
            pltpu.sync_copy(stg, out.at[c, p, pl.ds(r0, 16)])
            pltpu.sync_copy(zbuf, acc.at[pl.ds(r0, 16)])
            return 0
        lax.fori_loop(0, RPT // 16, _wb, 0)

    def round_(tab, rhbm, chbm, et, pr_row, pc_row, out, add_prev):
        pltpu.sync_copy(rhbm.at[pl.ds(t * et, et)], r_v.at[pl.ds(0, et)])
        pltpu.sync_copy(chbm.at[pl.ds(t * et, et)], c_v.at[pl.ds(0, et)])
        scalar_stage(et, pr_row, pc_row)

        def _pass(p, _):
            heavy(tab, et, p)
            plsc.subcore_barrier()
            writeback_zero(out, p, add_prev)
            plsc.subcore_barrier()
            return 0
        lax.fori_loop(0, NPASS, _pass, 0)

    round_(msgT, ai_h, aj_h, ET_A, 0, 1, out0, False)
    round_(smsgT, ti_h, sj_h, ET_I, 2, 3, out0, True)
    round_(tmsgT, sj_h, ti_h, ET_I, 5, 4, out2, False)


def kernel(x_0, x_2, adjacency_0, incidence_0_2, w_hbs, att_hbs, w_s, w_t, att_hbns):
    def cpad(c0, c1):
        z = jnp.zeros((256, 128), jnp.float32)
        z = z.at[:, 0].set(c0)
        return z.at[:, 1].set(c1)

    c_hbs = cpad(att_hbs[:256, 0], att_hbs[256:, 0])
    c_t = cpad(att_hbns[256:, 0], att_hbns[:256, 0])
    c_s = cpad(att_hbns[:256, 0], att_hbns[256:, 0])

    msg, pq = _mm_proj(x_0, w_hbs, c_hbs)
    t_msg, atbt = _mm_proj(x_0, w_t, c_t)
    s_msg, asbs = _mm_proj(x_2, w_s, c_s)

    npad = NP - N0

    def tabify(y):
        return jnp.pad(y, ((0, npad), (0, 0))).reshape(2 * NP, 128)

    def svec(v):
        return jnp.pad(v, (0, npad))

    scal = jnp.stack([svec(pq[:, 0]), svec(pq[:, 1]),
                      svec(atbt[:, 0]), svec(asbs[:, 0]),
                      svec(atbt[:, 1]), svec(asbs[:, 1])])

    ipad = NNZP - NNZ
    ti = jnp.pad(incidence_0_2[0], (0, ipad), constant_values=N0)
    sj = jnp.pad(incidence_0_2[1], (0, ipad), constant_values=N0)

    out0, out2 = _sparse_sc(tabify(msg), tabify(s_msg), tabify(t_msg), scal,
                            adjacency_0[0], adjacency_0[1], ti, sj)

    def assemble(o, n):
        cols = [o[cc].reshape(NPASS * NH, 128) for cc in (0, 1)]
        return jnp.concatenate(cols, axis=1)[:n]

    return (assemble(out0, N0), assemble(out2, N2))


# speedup vs baseline: 10.1799x; 1.0518x over previous
import functools

import jax
import jax.numpy as jnp
from jax import lax
from jax.experimental import pallas as pl
from jax.experimental.pallas import tpu as pltpu
from jax.experimental.pallas import tpu_sc as plsc

N0 = 10000
N2 = 10000
E = 160000
NNZ = 200000
NEG_SLOPE = 0.2

NP = 10240
NPR = NP // 128
NPASS = 2
NH = NP // NPASS
NHA = NH + 32
NNZP = 200192
ET_A = E // 16
ET_I = NNZP // 16
RPT = NH // 16
CAP = 1024
SCB = 62

_BLK = 1000

_mesh = plsc.VectorSubcoreMesh(core_axis_name="c", subcore_axis_name="s",
                               num_cores=2, num_subcores=16)
_CP = pltpu.CompilerParams(needs_layout_passes=False)


def _mm_body(x_ref, w_ref, c_ref, y_ref, pq_ref):
    y = jnp.dot(x_ref[...], w_ref[...], preferred_element_type=jnp.float32)
    y_ref[...] = y
    pq_ref[...] = jnp.dot(y, c_ref[...], preferred_element_type=jnp.float32)


def _mm_proj(x, w, c_pad):
    n, d_in = x.shape
    d_out = w.shape[1]
    return pl.pallas_call(
        _mm_body,
        grid=(n // _BLK,),
        in_specs=[
            pl.BlockSpec((_BLK, d_in), lambda i: (i, 0)),
            pl.BlockSpec((d_in, d_out), lambda i: (0, 0)),
            pl.BlockSpec((d_out, 128), lambda i: (0, 0)),
        ],
        out_specs=[
            pl.BlockSpec((_BLK, d_out), lambda i: (i, 0)),
            pl.BlockSpec((_BLK, 128), lambda i: (i, 0)),
        ],
        out_shape=[
            jax.ShapeDtypeStruct((n, d_out), jnp.float32),
            jax.ShapeDtypeStruct((n, 128), jnp.float32),
        ],
    )(x, w, c_pad)


@functools.partial(
    pl.kernel,
    out_type=[jax.ShapeDtypeStruct((2, NPASS, NH, 128), jnp.float32),
              jax.ShapeDtypeStruct((2, NPASS, NH, 128), jnp.float32)],
    mesh=_mesh,
    scratch_types=[
        pltpu.VMEM((ET_I,), jnp.int32),
        pltpu.VMEM((ET_I,), jnp.int32),
        pltpu.VMEM((ET_I,), jnp.float32),
        pltpu.VMEM((NP,), jnp.float32),
        pltpu.VMEM((NP,), jnp.float32),
        pltpu.VMEM((NPR, 128), jnp.float32),
        pltpu.VMEM((CAP,), jnp.int32),
        pltpu.VMEM((CAP,), jnp.int32),
        pltpu.VMEM((CAP,), jnp.float32),
        pltpu.VMEM((64, 128), jnp.float32),
        pltpu.VMEM((16, 128), jnp.float32),
        pltpu.VMEM((16, 128), jnp.float32),
        pltpu.VMEM((16, 128), jnp.float32),
        pltpu.VMEM((64,), jnp.int32),
        pltpu.VMEM((16,), jnp.int32),
        pltpu.VMEM((16,), jnp.int32),
        pltpu.VMEM((16,), jnp.int32),
        pltpu.VMEM((16,), jnp.int32),
        pltpu.VMEM((16,), jnp.int32),
        pltpu.VMEM((NPR,), jnp.int32),
        pltpu.VMEM_SHARED((NPR, 128), jnp.float32),
        pltpu.VMEM_SHARED((NHA, 128), jnp.float32),
        pltpu.SemaphoreType.DMA,
    ],
    compiler_params=_CP,
)
def _sparse_sc(msgT, smsgT, tmsgT, scal, ai_h, aj_h, ti_h, sj_h,
               out0, out2,
               r_v, c_v, att_v, pr_v, pc_v, sp_v, srid, scid, satt,
               rowbuf4, zbuf, stg, stg2,
               ridb4, ridbt, gidb, gidb1, gidb2, gidb3,
               rowids, s_sh, acc, sem):
    gidbs = (gidb, gidb1, gidb2, gidb3)
    c = lax.axis_index("c")
    t = lax.axis_index("s")
    zero16 = jnp.zeros((16,), jnp.float32)
    iota16 = lax.iota(jnp.int32, 16)
    nsr = NPR // 16

    def _zb(i, _):
        for v in range(8):
            zbuf[i, pl.ds(16 * v, 16)] = zero16
        return 0
    lax.fori_loop(0, 16, _zb, 0)

    def _fri(k, _):
        rowids[pl.ds(16 * k, 16)] = iota16 + 16 * k
        return 0
    lax.fori_loop(0, NPR // 16, _fri, 0)

    def _za(j, _):
        pltpu.sync_copy(zbuf, acc.at[pl.ds(t * RPT + 16 * j, 16)])
        return 0
    lax.fori_loop(0, RPT // 16, _za, 0)

    @pl.when(t == 0)
    def _():
        pltpu.sync_copy(zbuf, acc.at[pl.ds(NH, 16)])
        pltpu.sync_copy(zbuf, acc.at[pl.ds(NH + 16, 16)])

    pltpu.sync_copy(zbuf.at[pl.ds(0, nsr)], s_sh.at[pl.ds(nsr * t, nsr)])
    plsc.subcore_barrier()

    def scalar_stage(et, pr_row, pc_row):
        nch = et // 16
        pltpu.sync_copy(scal.at[pr_row], pr_v)
        pltpu.sync_copy(scal.at[pc_row], pc_v)

        def _zs(i, _):
            for v in range(8):
                sp_v[i, pl.ds(16 * v, 16)] = zero16
            return 0
        lax.fori_loop(0, NPR, _zs, 0)

        def _sta(k, _):
            rid = r_v[pl.ds(16 * k, 16)]
            cid = c_v[pl.ds(16 * k, 16)]
            e = (plsc.load_gather(pr_v, [rid])
                 + plsc.load_gather(pc_v, [cid]))
            e = jnp.where(e >= 0, e, NEG_SLOPE * e)
            ev = jnp.exp(e)
            att_v[pl.ds(16 * k, 16)] = ev
            plsc.addupdate_scatter(
                sp_v, [jnp.right_shift(rid, 7), jnp.bitwise_and(rid, 127)], ev)
            return 0
        lax.fori_loop(0, nch, _sta, 0)

        pltpu.sync_copy(sp_v, s_sh.at[rowids], add=True)
        plsc.subcore_barrier()
        pltpu.sync_copy(s_sh, sp_v)

        def _stb(k, _):
            rid = r_v[pl.ds(16 * k, 16)]
            sv = plsc.load_gather(
                sp_v, [jnp.right_shift(rid, 7), jnp.bitwise_and(rid, 127)])
            att = att_v[pl.ds(16 * k, 16)] / jnp.maximum(sv, 1e-30)
            att_v[pl.ds(16 * k, 16)] = att
            return 0
        lax.fori_loop(0, nch, _stb, 0)

        pltpu.sync_copy(zbuf.at[pl.ds(0, nsr)], s_sh.at[pl.ds(nsr * t, nsr)])
        plsc.subcore_barrier()

    def _drain_chunk(tab, k, _):
        rid = srid[pl.ds(16 * k, 16)]
        cid = scid[pl.ds(16 * k, 16)]
        ridbt[...] = rid
        gidb[...] = cid * 2 + c
        pltpu.async_copy(tab.at[gidb], rowbuf4.at[pl.ds(0, 16)], sem).wait()
        att_vec = satt[pl.ds(16 * k, 16)]
        for j in range(16):
            av = jnp.full((16,), att_vec[j], jnp.float32)
            for v in range(8):
                sl = pl.ds(16 * v, 16)
                rowbuf4[j, sl] = rowbuf4[j, sl] * av
        pltpu.sync_copy(rowbuf4.at[pl.ds(0, 16)], acc.at[ridbt], add=True)
        return 0

    def heavy(tab, et, p):
        nch = et // 16
        lo = p * NH

        def _scan(k, cur):
            rid = r_v[pl.ds(16 * k, 16)] - lo
            cid = c_v[pl.ds(16 * k, 16)]
            att = att_v[pl.ds(16 * k, 16)]
            mask = jnp.logical_and(rid >= 0, rid < NH)
            plsc.store_compressed(srid.at[pl.ds(cur, 16)], rid, mask=mask)
            plsc.store_compressed(scid.at[pl.ds(cur, 16)], cid, mask=mask)
            plsc.store_compressed(satt.at[pl.ds(cur, 16)], att, mask=mask)
            return cur + plsc.all_reduce_population_count(mask)[0]

        def _grp(b, _):
            base = 4 * b
            for s in range(4):
                k = base + s
                ridb4[pl.ds(16 * s, 16)] = srid[pl.ds(16 * k, 16)]
                gidbs[s][...] = scid[pl.ds(16 * k, 16)] * 2 + c
            descs = [pltpu.async_copy(tab.at[gidbs[s]],
                                      rowbuf4.at[pl.ds(16 * s, 16)], sem)
                     for s in range(4)]
            for s in range(4):
                k = base + s
                descs[s].wait()
                att_vec = satt[pl.ds(16 * k, 16)]
                for j in range(16):
                    av = jnp.full((16,), att_vec[j], jnp.float32)
                    for v in range(8):
                        sl = pl.ds(16 * v, 16)
                        rowbuf4[16 * s + j, sl] = rowbuf4[16 * s + j, sl] * av
            pltpu.sync_copy(rowbuf4, acc.at[ridb4], add=True)
            return 0

        def _drain(cur):
            full = cur // 16
            ngrp = full // 4
            lax.fori_loop(0, ngrp, _grp, 0)
            lax.fori_loop(4 * ngrp, full, functools.partial(_drain_chunk, tab), 0)
            rv = srid[pl.ds(16 * full, 16)]
            cv = scid[pl.ds(16 * full, 16)]
            av = satt[pl.ds(16 * full, 16)]
            srid[pl.ds(0, 16)] = rv
            scid[pl.ds(0, 16)] = cv
            satt[pl.ds(0, 16)] = av
            return cur - full * 16

        nblk = (nch + SCB - 1) // SCB

        def _blk(b, cur):
            start = b * SCB
            end = jnp.minimum(start + SCB, nch)
            cur = lax.fori_loop(start, end, _scan, cur)
            return _drain(cur)

        cur = lax.fori_loop(0, nblk, _blk, jnp.int32(0))
        rv = srid[pl.ds(0, 16)]
        cv = scid[pl.ds(0, 16)]
        av = satt[pl.ds(0, 16)]
        live = iota16 < cur
        srid[pl.ds(0, 16)] = jnp.where(live, rv, NH)
        scid[pl.ds(0, 16)] = jnp.where(live, cv, 0)
        satt[pl.ds(0, 16)] = jnp.where(live, av, 0.0)
        _drain_chunk(tab, 0, 0)

    def writeback_zero(out, p, add_prev):
        def _wb(j, _):
            r0 = t * RPT + 16 * j
            pltpu.sync_copy(acc.at[pl.ds(r0, 16)], stg)
            if add_prev:
                pltpu.sync_copy(out.at[c, p, pl.ds(r0, 16)], stg2)

                def _addrow(i, _):
                    for v in range(8):
                        sl = pl.ds(16 * v, 16)
                        stg[i, sl] = stg[i, sl] + stg2[i, sl]
                    return 0
                lax.fori_loop(0, 16, _addrow, 0)
            pltpu.sync_copy(stg, out.at[c, p, pl.ds(r0, 16)])
            pltpu.sync_copy(zbuf, acc.at[pl.ds(r0, 16)])
            return 0
        lax.fori_loop(0, RPT // 16, _wb, 0)

    def round_(tab, rhbm, chbm, et, pr_row, pc_row, out, add_prev):
        pltpu.sync_copy(rhbm.at[pl.ds(t * et, et)], r_v.at[pl.ds(0, et)])
        pltpu.sync_copy(chbm.at[pl.ds(t * et, et)], c_v.at[pl.ds(0, et)])
        scalar_stage(et, pr_row, pc_row)

        def _pass(p, _):
            heavy(tab, et, p)
            plsc.subcore_barrier()
            writeback_zero(out, p, add_prev)
            plsc.subcore_barrier()
            return 0
        lax.fori_loop(0, NPASS, _pass, 0)

    round_(msgT, ai_h, aj_h, ET_A, 0, 1, out0, False)
    round_(smsgT, ti_h, sj_h, ET_I, 2, 3, out0, True)
    round_(tmsgT, sj_h, ti_h, ET_I, 5, 4, out2, False)


def kernel(x_0, x_2, adjacency_0, incidence_0_2, w_hbs, att_hbs, w_s, w_t, att_hbns):
    def cpad(c0, c1):
        z = jnp.zeros((256, 128), jnp.float32)
        z = z.at[:, 0].set(c0)
        return z.at[:, 1].set(c1)

    c_hbs = cpad(att_hbs[:256, 0], att_hbs[256:, 0])
    c_t = cpad(att_hbns[256:, 0], att_hbns[:256, 0])
    c_s = cpad(att_hbns[:256, 0], att_hbns[256:, 0])

    msg, pq = _mm_proj(x_0, w_hbs, c_hbs)
    t_msg, atbt = _mm_proj(x_0, w_t, c_t)
    s_msg, asbs = _mm_proj(x_2, w_s, c_s)

    npad = NP - N0

    def tabify(y):
        return jnp.pad(y, ((0, npad), (0, 0))).reshape(2 * NP, 128)

    def svec(v):
        return jnp.pad(v, (0, npad))

    scal = jnp.stack([svec(pq[:, 0]), svec(pq[:, 1]),
                      svec(atbt[:, 0]), svec(asbs[:, 0]),
                      svec(atbt[:, 1]), svec(asbs[:, 1])])

    ipad = NNZP - NNZ
    ti = jnp.pad(incidence_0_2[0], (0, ipad), constant_values=N0)
    sj = jnp.pad(incidence_0_2[1], (0, ipad), constant_values=N0)

    out0, out2 = _sparse_sc(tabify(msg), tabify(s_msg), tabify(t_msg), scal,
                            adjacency_0[0], adjacency_0[1], ti, sj)

    def assemble(o, n):
        cols = [o[cc].reshape(NPASS * NH, 128) for cc in (0, 1)]
        return jnp.concatenate(cols, axis=1)[:n]

    return (assemble(out0, N0), assemble(out2, N2))


# speedup vs baseline: 11.8138x; 1.1605x over previous
import functools

import jax
import jax.numpy as jnp
from jax import lax
from jax.experimental import pallas as pl
from jax.experimental.pallas import tpu as pltpu
from jax.experimental.pallas import tpu_sc as plsc

N0 = 10000
N2 = 10000
E = 160000
NNZ = 200000
NEG_SLOPE = 0.2

NP = 10240
NPR = NP // 128
NPASS = 2
NH = NP // NPASS
NHA = NH + 32
NNZP = 200192
ET_A = E // 16
ET_I = NNZP // 16
RPT = NH // 16
CAP = 1024
SCB = 62

_BLK = 1000

_mesh = plsc.VectorSubcoreMesh(core_axis_name="c", subcore_axis_name="s",
                               num_cores=2, num_subcores=16)
_CP = pltpu.CompilerParams(needs_layout_passes=False)


def _mm_body(x_ref, w_ref, c_ref, y_ref, pq_ref):
    y = jnp.dot(x_ref[...], w_ref[...], preferred_element_type=jnp.float32)
    y_ref[...] = y
    pq_ref[...] = jnp.dot(y, c_ref[...], preferred_element_type=jnp.float32)


def _mm_proj(x, w, c_pad):
    n, d_in = x.shape
    d_out = w.shape[1]
    return pl.pallas_call(
        _mm_body,
        grid=(n // _BLK,),
        in_specs=[
            pl.BlockSpec((_BLK, d_in), lambda i: (i, 0)),
            pl.BlockSpec((d_in, d_out), lambda i: (0, 0)),
            pl.BlockSpec((d_out, 128), lambda i: (0, 0)),
        ],
        out_specs=[
            pl.BlockSpec((_BLK, d_out), lambda i: (i, 0)),
            pl.BlockSpec((_BLK, 128), lambda i: (i, 0)),
        ],
        out_shape=[
            jax.ShapeDtypeStruct((n, d_out), jnp.float32),
            jax.ShapeDtypeStruct((n, 128), jnp.float32),
        ],
    )(x, w, c_pad)


@functools.partial(
    pl.kernel,
    out_type=[jax.ShapeDtypeStruct((2, NPASS, NH, 128), jnp.float32),
              jax.ShapeDtypeStruct((2, NPASS, NH, 128), jnp.float32)],
    mesh=_mesh,
    scratch_types=[
        pltpu.VMEM((ET_I,), jnp.int32),
        pltpu.VMEM((ET_I,), jnp.int32),
        pltpu.VMEM((ET_I,), jnp.float32),
        pltpu.VMEM((NP,), jnp.float32),
        pltpu.VMEM((NP,), jnp.float32),
        pltpu.VMEM((NPR, 128), jnp.float32),
        pltpu.VMEM((CAP,), jnp.int32),
        pltpu.VMEM((CAP,), jnp.int32),
        pltpu.VMEM((CAP,), jnp.float32),
        pltpu.VMEM((64, 128), jnp.float32),
        pltpu.VMEM((64, 128), jnp.float32),
        pltpu.VMEM((64,), jnp.int32),
        pltpu.VMEM((64,), jnp.int32),
        pltpu.VMEM((64,), jnp.int32),
        pltpu.VMEM((64,), jnp.int32),
        pltpu.VMEM((16,), jnp.int32),
        pltpu.VMEM((16,), jnp.int32),
        pltpu.VMEM((NPR,), jnp.int32),
        pltpu.VMEM_SHARED((NPR, 128), jnp.float32),
        pltpu.VMEM_SHARED((NHA, 128), jnp.float32),
        pltpu.SemaphoreType.DMA,
        pltpu.SemaphoreType.DMA,
    ],
    compiler_params=_CP,
)
def _sparse_sc(msgT, smsgT, tmsgT, scal, ai_h, aj_h, ti_h, sj_h,
               out0, out2,
               r_v, c_v, att_v, pr_v, pc_v, sp_v, srid, scid, satt,
               rowbuf4, rowbuf4b,
               ridb4, ridb4b, gidb4, gidb4b, ridbt, gidb,
               rowids, s_sh, acc, semA, semB):
    zbuf = rowbuf4b.at[pl.ds(0, 16)]
    stg = rowbuf4b.at[pl.ds(16, 16)]
    stg2 = rowbuf4b.at[pl.ds(32, 16)]
    c = lax.axis_index("c")
    t = lax.axis_index("s")
    zero16 = jnp.zeros((16,), jnp.float32)
    iota16 = lax.iota(jnp.int32, 16)
    nsr = NPR // 16

    def _fill_zeros():
        def _zb(i, _):
            for v in range(8):
                rowbuf4b[i, pl.ds(16 * v, 16)] = zero16
            return 0
        lax.fori_loop(0, 16, _zb, 0)

    _fill_zeros()

    def _fri(k, _):
        rowids[pl.ds(16 * k, 16)] = iota16 + 16 * k
        return 0
    lax.fori_loop(0, NPR // 16, _fri, 0)

    def _za(j, _):
        pltpu.sync_copy(zbuf, acc.at[pl.ds(t * RPT + 16 * j, 16)])
        return 0
    lax.fori_loop(0, RPT // 16, _za, 0)

    @pl.when(t == 0)
    def _():
        pltpu.sync_copy(zbuf, acc.at[pl.ds(NH, 16)])
        pltpu.sync_copy(zbuf, acc.at[pl.ds(NH + 16, 16)])

    pltpu.sync_copy(rowbuf4b.at[pl.ds(0, nsr)], s_sh.at[pl.ds(nsr * t, nsr)])
    plsc.subcore_barrier()

    def scalar_stage(et, pr_row, pc_row):
        nch = et // 16
        pltpu.sync_copy(scal.at[pr_row], pr_v)
        pltpu.sync_copy(scal.at[pc_row], pc_v)

        def _zs(i, _):
            for v in range(8):
                sp_v[i, pl.ds(16 * v, 16)] = zero16
            return 0
        lax.fori_loop(0, NPR, _zs, 0)

        def _sta(k, _):
            rid = r_v[pl.ds(16 * k, 16)]
            cid = c_v[pl.ds(16 * k, 16)]
            e = (plsc.load_gather(pr_v, [rid])
                 + plsc.load_gather(pc_v, [cid]))
            e = jnp.where(e >= 0, e, NEG_SLOPE * e)
            ev = jnp.exp(e)
            att_v[pl.ds(16 * k, 16)] = ev
            plsc.addupdate_scatter(
                sp_v, [jnp.right_shift(rid, 7), jnp.bitwise_and(rid, 127)], ev)
            return 0
        lax.fori_loop(0, nch, _sta, 0)

        pltpu.sync_copy(sp_v, s_sh.at[rowids], add=True)
        plsc.subcore_barrier()
        pltpu.sync_copy(s_sh, sp_v)

        def _stb(k, _):
            rid = r_v[pl.ds(16 * k, 16)]
            sv = plsc.load_gather(
                sp_v, [jnp.right_shift(rid, 7), jnp.bitwise_and(rid, 127)])
            att = att_v[pl.ds(16 * k, 16)] / jnp.maximum(sv, 1e-30)
            att_v[pl.ds(16 * k, 16)] = att
            return 0
        lax.fori_loop(0, nch, _stb, 0)

        _fill_zeros()
        pltpu.sync_copy(rowbuf4b.at[pl.ds(0, nsr)],
                        s_sh.at[pl.ds(nsr * t, nsr)])
        plsc.subcore_barrier()

    def _drain_chunk(tab, k, _):
        rid = srid[pl.ds(16 * k, 16)]
        cid = scid[pl.ds(16 * k, 16)]
        ridbt[...] = rid
        gidb[...] = cid * 2 + c
        pltpu.async_copy(tab.at[gidb], rowbuf4.at[pl.ds(0, 16)], semA).wait()
        att_vec = satt[pl.ds(16 * k, 16)]
        for j in range(16):
            av = jnp.full((16,), att_vec[j], jnp.float32)
            for v in range(8):
                sl = pl.ds(16 * v, 16)
                rowbuf4[j, sl] = rowbuf4[j, sl] * av
        pltpu.sync_copy(rowbuf4.at[pl.ds(0, 16)], acc.at[ridbt], add=True)
        return 0

    def heavy(tab, et, p):
        nch = et // 16
        lo = p * NH

        def _scan(k, cur):
            rid = r_v[pl.ds(16 * k, 16)] - lo
            cid = c_v[pl.ds(16 * k, 16)]
            att = att_v[pl.ds(16 * k, 16)]
            mask = jnp.logical_and(rid >= 0, rid < NH)
            plsc.store_compressed(srid.at[pl.ds(cur, 16)], rid, mask=mask)
            plsc.store_compressed(scid.at[pl.ds(cur, 16)], cid, mask=mask)
            plsc.store_compressed(satt.at[pl.ds(cur, 16)], att, mask=mask)
            return cur + plsc.all_reduce_population_count(mask)[0]

        def _prep(g, ridbX, gidbX, rowbufX, semX):
            base = 64 * g
            for s in range(4):
                ridbX[pl.ds(16 * s, 16)] = srid[pl.ds(base + 16 * s, 16)]
                gidbX[pl.ds(16 * s, 16)] = (
                    scid[pl.ds(base + 16 * s, 16)] * 2 + c)
            pltpu.async_copy(tab.at[gidbX], rowbufX, semX)

        def _wait(rowbufX, semX):
            pltpu.make_async_copy(msgT.at[pl.ds(0, 64)], rowbufX, semX).wait()

        def _scale_scatter(g, ridbX, rowbufX):
            for s in range(4):
                att_vec = satt[pl.ds(64 * g + 16 * s, 16)]
                for j in range(16):
                    av = jnp.full((16,), att_vec[j], jnp.float32)
                    for v in range(8):
                        sl = pl.ds(16 * v, 16)
                        rowbufX[16 * s + j, sl] = rowbufX[16 * s + j, sl] * av
            pltpu.sync_copy(rowbufX, acc.at[ridbX], add=True)

        def _drain(cur):
            full = cur // 16
            ngrp = full // 4

            @pl.when(ngrp > 0)
            def _():
                _prep(0, ridb4, gidb4, rowbuf4, semA)

            def _pair(i, _):
                ga = 2 * i
                gb = ga + 1
                _wait(rowbuf4, semA)

                @pl.when(gb < ngrp)
                def _():
                    _prep(gb, ridb4b, gidb4b, rowbuf4b, semB)
                _scale_scatter(ga, ridb4, rowbuf4)

                @pl.when(gb < ngrp)
                def _():
                    _wait(rowbuf4b, semB)

                    @pl.when(gb + 1 < ngrp)
                    def _():
                        _prep(gb + 1, ridb4, gidb4, rowbuf4, semA)
                    _scale_scatter(gb, ridb4b, rowbuf4b)
                return 0
            lax.fori_loop(0, (ngrp + 1) // 2, _pair, 0)
            lax.fori_loop(4 * ngrp, full, functools.partial(_drain_chunk, tab), 0)
            rv = srid[pl.ds(16 * full, 16)]
            cv = scid[pl.ds(16 * full, 16)]
            av = satt[pl.ds(16 * full, 16)]
            srid[pl.ds(0, 16)] = rv
            scid[pl.ds(0, 16)] = cv
            satt[pl.ds(0, 16)] = av
            return cur - full * 16

        nblk = (nch + SCB - 1) // SCB

        def _blk(b, cur):
            start = b * SCB
            end = jnp.minimum(start + SCB, nch)
            cur = lax.fori_loop(start, end, _scan, cur)
            return _drain(cur)

        cur = lax.fori_loop(0, nblk, _blk, jnp.int32(0))
        rv = srid[pl.ds(0, 16)]
        cv = scid[pl.ds(0, 16)]
        av = satt[pl.ds(0, 16)]
        live = iota16 < cur
        srid[pl.ds(0, 16)] = jnp.where(live, rv, NH)
        scid[pl.ds(0, 16)] = jnp.where(live, cv, 0)
        satt[pl.ds(0, 16)] = jnp.where(live, av, 0.0)
        _drain_chunk(tab, 0, 0)

    def writeback_zero(out, p, add_prev):
        _fill_zeros()

        def _wb(j, _):
            r0 = t * RPT + 16 * j
            pltpu.sync_copy(acc.at[pl.ds(r0, 16)], stg)
            if add_prev:
                pltpu.sync_copy(out.at[c, p, pl.ds(r0, 16)], stg2)

                def _addrow(i, _):
                    for v in range(8):
                        sl = pl.ds(16 * v, 16)
                        rowbuf4b[16 + i, sl] = (rowbuf4b[16 + i, sl]
                                                + rowbuf4b[32 + i, sl])
                    return 0
                lax.fori_loop(0, 16, _addrow, 0)
            pltpu.sync_copy(stg, out.at[c, p, pl.ds(r0, 16)])
            pltpu.sync_copy(zbuf, acc.at[pl.ds(r0, 16)])
            return 0
        lax.fori_loop(0, RPT // 16, _wb, 0)

    def round_(tab, rhbm, chbm, et, pr_row, pc_row, out, add_prev):
        pltpu.sync_copy(rhbm.at[pl.ds(t * et, et)], r_v.at[pl.ds(0, et)])
        pltpu.sync_copy(chbm.at[pl.ds(t * et, et)], c_v.at[pl.ds(0, et)])
        scalar_stage(et, pr_row, pc_row)

        def _pass(p, _):
            heavy(tab, et, p)
            plsc.subcore_barrier()
            writeback_zero(out, p, add_prev)
            plsc.subcore_barrier()
            return 0
        lax.fori_loop(0, NPASS, _pass, 0)

    round_(msgT, ai_h, aj_h, ET_A, 0, 1, out0, False)
    round_(smsgT, ti_h, sj_h, ET_I, 2, 3, out0, True)
    round_(tmsgT, sj_h, ti_h, ET_I, 5, 4, out2, False)


def kernel(x_0, x_2, adjacency_0, incidence_0_2, w_hbs, att_hbs, w_s, w_t, att_hbns):
    def cpad(c0, c1):
        z = jnp.zeros((256, 128), jnp.float32)
        z = z.at[:, 0].set(c0)
        return z.at[:, 1].set(c1)

    c_hbs = cpad(att_hbs[:256, 0], att_hbs[256:, 0])
    c_t = cpad(att_hbns[256:, 0], att_hbns[:256, 0])
    c_s = cpad(att_hbns[:256, 0], att_hbns[256:, 0])

    msg, pq = _mm_proj(x_0, w_hbs, c_hbs)
    t_msg, atbt = _mm_proj(x_0, w_t, c_t)
    s_msg, asbs = _mm_proj(x_2, w_s, c_s)

    npad = NP - N0

    def tabify(y):
        return jnp.pad(y, ((0, npad), (0, 0))).reshape(2 * NP, 128)

    def svec(v):
        return jnp.pad(v, (0, npad))

    scal = jnp.stack([svec(pq[:, 0]), svec(pq[:, 1]),
                      svec(atbt[:, 0]), svec(asbs[:, 0]),
                      svec(atbt[:, 1]), svec(asbs[:, 1])])

    ipad = NNZP - NNZ
    ti = jnp.pad(incidence_0_2[0], (0, ipad), constant_values=N0)
    sj = jnp.pad(incidence_0_2[1], (0, ipad), constant_values=N0)

    out0, out2 = _sparse_sc(tabify(msg), tabify(s_msg), tabify(t_msg), scal,
                            adjacency_0[0], adjacency_0[1], ti, sj)

    def assemble(o, n):
        cols = [o[cc].reshape(NPASS * NH, 128) for cc in (0, 1)]
        return jnp.concatenate(cols, axis=1)[:n]

    return (assemble(out0, N0), assemble(out2, N2))


# speedup vs baseline: 12.5744x; 1.0644x over previous
import functools

import jax
import jax.numpy as jnp
from jax import lax
from jax.experimental import pallas as pl
from jax.experimental.pallas import tpu as pltpu
from jax.experimental.pallas import tpu_sc as plsc

N0 = 10000
N2 = 10000
E = 160000
NNZ = 200000
NEG_SLOPE = 0.2

NP = 10240
NPR = NP // 128
NPASS = 2
NH = NP // NPASS
NHA = NH + 32
NNZP = 200192
ET_A = E // 16
ET_I = NNZP // 16
RPT = NH // 16
CAP = 1024
SCB = 59

_BLK = 1000

_mesh = plsc.VectorSubcoreMesh(core_axis_name="c", subcore_axis_name="s",
                               num_cores=2, num_subcores=16)
_CP = pltpu.CompilerParams(needs_layout_passes=False)


def _mm_body(x_ref, w_ref, c_ref, y_ref, pq_ref):
    y = jnp.dot(x_ref[...], w_ref[...], preferred_element_type=jnp.float32)
    y_ref[...] = y
    pq_ref[...] = jnp.dot(y, c_ref[...], preferred_element_type=jnp.float32)


def _mm_proj(x, w, c_pad):
    n, d_in = x.shape
    d_out = w.shape[1]
    return pl.pallas_call(
        _mm_body,
        grid=(n // _BLK,),
        in_specs=[
            pl.BlockSpec((_BLK, d_in), lambda i: (i, 0)),
            pl.BlockSpec((d_in, d_out), lambda i: (0, 0)),
            pl.BlockSpec((d_out, 128), lambda i: (0, 0)),
        ],
        out_specs=[
            pl.BlockSpec((_BLK, d_out), lambda i: (i, 0)),
            pl.BlockSpec((_BLK, 128), lambda i: (i, 0)),
        ],
        out_shape=[
            jax.ShapeDtypeStruct((n, d_out), jnp.float32),
            jax.ShapeDtypeStruct((n, 128), jnp.float32),
        ],
    )(x, w, c_pad)


@functools.partial(
    pl.kernel,
    out_type=[jax.ShapeDtypeStruct((2, NPASS, NH, 128), jnp.float32),
              jax.ShapeDtypeStruct((2, NPASS, NH, 128), jnp.float32)],
    mesh=_mesh,
    scratch_types=[
        pltpu.VMEM((ET_I,), jnp.int32),
        pltpu.VMEM((ET_I,), jnp.int32),
        pltpu.VMEM((ET_I,), jnp.float32),
        pltpu.VMEM((NP,), jnp.float32),
        pltpu.VMEM((NP,), jnp.float32),
        pltpu.VMEM((NPR, 128), jnp.float32),
        pltpu.VMEM((CAP,), jnp.int32),
        pltpu.VMEM((CAP,), jnp.int32),
        pltpu.VMEM((CAP,), jnp.float32),
        pltpu.VMEM((64, 128), jnp.float32),
        pltpu.VMEM((64, 128), jnp.float32),
        pltpu.VMEM((64,), jnp.int32),
        pltpu.VMEM((64,), jnp.int32),
        pltpu.VMEM((64,), jnp.int32),
        pltpu.VMEM((64,), jnp.int32),
        pltpu.VMEM((16,), jnp.int32),
        pltpu.VMEM((16,), jnp.int32),
        pltpu.VMEM((NPR,), jnp.int32),
        pltpu.VMEM_SHARED((NPR, 128), jnp.float32),
        pltpu.VMEM_SHARED((NHA, 128), jnp.float32),
        pltpu.SemaphoreType.DMA,
        pltpu.SemaphoreType.DMA,
    ],
    compiler_params=_CP,
)
def _sparse_sc(msgT, smsgT, tmsgT, scal, ai_h, aj_h, ti_h, sj_h,
               out0, out2,
               r_v, c_v, att_v, pr_v, pc_v, sp_v, srid, scid, satt,
               rowbuf4, rowbuf4b,
               ridb4, ridb4b, gidb4, gidb4b, ridbt, gidb,
               rowids, s_sh, acc, semA, semB):
    zbuf = rowbuf4b.at[pl.ds(0, 16)]
    stg = rowbuf4b.at[pl.ds(16, 16)]
    stg2 = rowbuf4b.at[pl.ds(32, 16)]
    c = lax.axis_index("c")
    t = lax.axis_index("s")
    zero16 = jnp.zeros((16,), jnp.float32)
    iota16 = lax.iota(jnp.int32, 16)
    nsr = NPR // 16

    def _fill_zeros():
        def _zb(i, _):
            for v in range(8):
                rowbuf4b[i, pl.ds(16 * v, 16)] = zero16
            return 0
        lax.fori_loop(0, 16, _zb, 0)

    _fill_zeros()

    def _fri(k, _):
        rowids[pl.ds(16 * k, 16)] = iota16 + 16 * k
        return 0
    lax.fori_loop(0, NPR // 16, _fri, 0)

    def _za(j, _):
        pltpu.sync_copy(zbuf, acc.at[pl.ds(t * RPT + 16 * j, 16)])
        return 0
    lax.fori_loop(0, RPT // 16, _za, 0)

    @pl.when(t == 0)
    def _():
        pltpu.sync_copy(zbuf, acc.at[pl.ds(NH, 16)])
        pltpu.sync_copy(zbuf, acc.at[pl.ds(NH + 16, 16)])

    pltpu.sync_copy(rowbuf4b.at[pl.ds(0, nsr)], s_sh.at[pl.ds(nsr * t, nsr)])
    plsc.subcore_barrier()

    def scalar_stage(et, pr_row, pc_row):
        nch = et // 16
        pltpu.sync_copy(scal.at[pr_row], pr_v)
        pltpu.sync_copy(scal.at[pc_row], pc_v)

        def _zs(i, _):
            for v in range(8):
                sp_v[i, pl.ds(16 * v, 16)] = zero16
            return 0
        lax.fori_loop(0, NPR, _zs, 0)

        def _sta(k, _):
            rid = r_v[pl.ds(16 * k, 16)]
            cid = c_v[pl.ds(16 * k, 16)]
            e = (plsc.load_gather(pr_v, [rid])
                 + plsc.load_gather(pc_v, [cid]))
            e = jnp.where(e >= 0, e, NEG_SLOPE * e)
            ev = jnp.exp(e)
            att_v[pl.ds(16 * k, 16)] = ev
            plsc.addupdate_scatter(
                sp_v, [jnp.right_shift(rid, 7), jnp.bitwise_and(rid, 127)], ev)
            return 0
        lax.fori_loop(0, nch, _sta, 0)

        pltpu.sync_copy(sp_v, s_sh.at[rowids], add=True)
        plsc.subcore_barrier()
        pltpu.sync_copy(s_sh, sp_v)

        def _stb(k, _):
            rid = r_v[pl.ds(16 * k, 16)]
            sv = plsc.load_gather(
                sp_v, [jnp.right_shift(rid, 7), jnp.bitwise_and(rid, 127)])
            att = att_v[pl.ds(16 * k, 16)] / jnp.maximum(sv, 1e-30)
            att_v[pl.ds(16 * k, 16)] = att
            return 0
        lax.fori_loop(0, nch, _stb, 0)

        _fill_zeros()
        pltpu.sync_copy(rowbuf4b.at[pl.ds(0, nsr)],
                        s_sh.at[pl.ds(nsr * t, nsr)])
        plsc.subcore_barrier()

    def _drain_chunk(tab, k, _):
        rid = srid[pl.ds(16 * k, 16)]
        cid = scid[pl.ds(16 * k, 16)]
        ridbt[...] = rid
        gidb[...] = cid * 2 + c
        pltpu.async_copy(tab.at[gidb], rowbuf4.at[pl.ds(0, 16)], semA).wait()
        att_vec = satt[pl.ds(16 * k, 16)]
        for j in range(16):
            av = jnp.full((16,), att_vec[j], jnp.float32)
            for v in range(8):
                sl = pl.ds(16 * v, 16)
                rowbuf4[j, sl] = rowbuf4[j, sl] * av
        pltpu.sync_copy(rowbuf4.at[pl.ds(0, 16)], acc.at[ridbt], add=True)
        return 0

    def heavy(tab, et, p):
        nch = et // 16
        lo = p * NH

        def _scan(k, cur):
            rid = r_v[pl.ds(16 * k, 16)] - lo
            cid = c_v[pl.ds(16 * k, 16)]
            att = att_v[pl.ds(16 * k, 16)]
            mask = jnp.logical_and(rid >= 0, rid < NH)
            plsc.store_compressed(srid.at[pl.ds(cur, 16)], rid, mask=mask)
            plsc.store_compressed(scid.at[pl.ds(cur, 16)], cid, mask=mask)
            plsc.store_compressed(satt.at[pl.ds(cur, 16)], att, mask=mask)
            return cur + plsc.all_reduce_population_count(mask)[0]

        def _prep(g, ridbX, gidbX, rowbufX, semX):
            base = 64 * g
            for s in range(4):
                ridbX[pl.ds(16 * s, 16)] = srid[pl.ds(base + 16 * s, 16)]
                gidbX[pl.ds(16 * s, 16)] = (
                    scid[pl.ds(base + 16 * s, 16)] * 2 + c)
            pltpu.async_copy(tab.at[gidbX], rowbufX, semX)

        def _wait(rowbufX, semX):
            pltpu.make_async_copy(msgT.at[pl.ds(0, 64)], rowbufX, semX).wait()

        def _scale_scatter(g, ridbX, rowbufX):
            for s in range(4):
                att_vec = satt[pl.ds(64 * g + 16 * s, 16)]
                for j in range(16):
                    av = jnp.full((16,), att_vec[j], jnp.float32)
                    for v in range(8):
                        sl = pl.ds(16 * v, 16)
                        rowbufX[16 * s + j, sl] = rowbufX[16 * s + j, sl] * av
            pltpu.sync_copy(rowbufX, acc.at[ridbX], add=True)

        def _drain(cur):
            ngrp = cur // 64

            @pl.when(ngrp > 0)
            def _():
                _prep(0, ridb4, gidb4, rowbuf4, semA)

            def _pair(i, _):
                ga = 2 * i
                gb = ga + 1
                _wait(rowbuf4, semA)

                @pl.when(gb < ngrp)
                def _():
                    _prep(gb, ridb4b, gidb4b, rowbuf4b, semB)
                _scale_scatter(ga, ridb4, rowbuf4)

                @pl.when(gb < ngrp)
                def _():
                    _wait(rowbuf4b, semB)

                    @pl.when(gb + 1 < ngrp)
                    def _():
                        _prep(gb + 1, ridb4, gidb4, rowbuf4, semA)
                    _scale_scatter(gb, ridb4b, rowbuf4b)
                return 0
            lax.fori_loop(0, (ngrp + 1) // 2, _pair, 0)
            base = 64 * ngrp
            for s in range(4):
                rv = srid[pl.ds(base + 16 * s, 16)]
                cv = scid[pl.ds(base + 16 * s, 16)]
                av = satt[pl.ds(base + 16 * s, 16)]
                srid[pl.ds(16 * s, 16)] = rv
                scid[pl.ds(16 * s, 16)] = cv
                satt[pl.ds(16 * s, 16)] = av
            return cur - base

        nblk = (nch + SCB - 1) // SCB

        def _blk(b, cur):
            start = b * SCB
            end = jnp.minimum(start + SCB, nch)
            cur = lax.fori_loop(start, end, _scan, cur)
            return _drain(cur)

        cur = lax.fori_loop(0, nblk, _blk, jnp.int32(0))
        fullt = cur // 16
        lax.fori_loop(0, fullt, functools.partial(_drain_chunk, tab), 0)
        rem = cur - fullt * 16
        rv = srid[pl.ds(16 * fullt, 16)]
        cv = scid[pl.ds(16 * fullt, 16)]
        av = satt[pl.ds(16 * fullt, 16)]
        live = iota16 < rem
        srid[pl.ds(0, 16)] = jnp.where(live, rv, NH)
        scid[pl.ds(0, 16)] = jnp.where(live, cv, 0)
        satt[pl.ds(0, 16)] = jnp.where(live, av, 0.0)
        _drain_chunk(tab, 0, 0)

    def writeback_zero(out, p, add_prev):
        _fill_zeros()

        def _wb(j, _):
            r0 = t * RPT + 16 * j
            pltpu.sync_copy(acc.at[pl.ds(r0, 16)], stg)
            if add_prev:
                pltpu.sync_copy(out.at[c, p, pl.ds(r0, 16)], stg2)

                def _addrow(i, _):
                    for v in range(8):
                        sl = pl.ds(16 * v, 16)
                        rowbuf4b[16 + i, sl] = (rowbuf4b[16 + i, sl]
                                                + rowbuf4b[32 + i, sl])
                    return 0
                lax.fori_loop(0, 16, _addrow, 0)
            pltpu.sync_copy(stg, out.at[c, p, pl.ds(r0, 16)])
            pltpu.sync_copy(zbuf, acc.at[pl.ds(r0, 16)])
            return 0
        lax.fori_loop(0, RPT // 16, _wb, 0)

    def round_(tab, rhbm, chbm, et, pr_row, pc_row, out, add_prev):
        pltpu.sync_copy(rhbm.at[pl.ds(t * et, et)], r_v.at[pl.ds(0, et)])
        pltpu.sync_copy(chbm.at[pl.ds(t * et, et)], c_v.at[pl.ds(0, et)])
        scalar_stage(et, pr_row, pc_row)

        def _pass(p, _):
            heavy(tab, et, p)
            plsc.subcore_barrier()
            writeback_zero(out, p, add_prev)
            plsc.subcore_barrier()
            return 0
        lax.fori_loop(0, NPASS, _pass, 0)

    round_(msgT, ai_h, aj_h, ET_A, 0, 1, out0, False)
    round_(smsgT, ti_h, sj_h, ET_I, 2, 3, out0, True)
    round_(tmsgT, sj_h, ti_h, ET_I, 5, 4, out2, False)


def kernel(x_0, x_2, adjacency_0, incidence_0_2, w_hbs, att_hbs, w_s, w_t, att_hbns):
    def cpad(c0, c1):
        z = jnp.zeros((256, 128), jnp.float32)
        z = z.at[:, 0].set(c0)
        return z.at[:, 1].set(c1)

    c_hbs = cpad(att_hbs[:256, 0], att_hbs[256:, 0])
    c_t = cpad(att_hbns[256:, 0], att_hbns[:256, 0])
    c_s = cpad(att_hbns[:256, 0], att_hbns[256:, 0])

    msg, pq = _mm_proj(x_0, w_hbs, c_hbs)
    t_msg, atbt = _mm_proj(x_0, w_t, c_t)
    s_msg, asbs = _mm_proj(x_2, w_s, c_s)

    npad = NP - N0

    def tabify(y):
        return jnp.pad(y, ((0, npad), (0, 0))).reshape(2 * NP, 128)

    def svec(v):
        return jnp.pad(v, (0, npad))

    scal = jnp.stack([svec(pq[:, 0]), svec(pq[:, 1]),
                      svec(atbt[:, 0]), svec(asbs[:, 0]),
                      svec(atbt[:, 1]), svec(asbs[:, 1])])

    ipad = NNZP - NNZ
    ti = jnp.pad(incidence_0_2[0], (0, ipad), constant_values=N0)
    sj = jnp.pad(incidence_0_2[1], (0, ipad), constant_values=N0)

    out0, out2 = _sparse_sc(tabify(msg), tabify(s_msg), tabify(t_msg), scal,
                            adjacency_0[0], adjacency_0[1], ti, sj)

    def assemble(o, n):
        cols = [o[cc].reshape(NPASS * NH, 128) for cc in (0, 1)]
        return jnp.concatenate(cols, axis=1)[:n]

    return (assemble(out0, N0), assemble(out2, N2))


# speedup vs baseline: 12.9817x; 1.0324x over previous
import functools

import jax
import jax.numpy as jnp
from jax import lax
from jax.experimental import pallas as pl
from jax.experimental.pallas import tpu as pltpu
from jax.experimental.pallas import tpu_sc as plsc

N0 = 10000
N2 = 10000
E = 160000
NNZ = 200000
NEG_SLOPE = 0.2

NP = 10240
NPR = NP // 128
NPASS = 2
NH = NP // NPASS
NHA = NH + 32
NNZP = 200192
ET_A = E // 16
ET_I = NNZP // 16
RPT = NH // 16
CAP = 1024
SCB = 61

_BLK = 1000

_mesh = plsc.VectorSubcoreMesh(core_axis_name="c", subcore_axis_name="s",
                               num_cores=2, num_subcores=16)
_CP = pltpu.CompilerParams(needs_layout_passes=False)


def _mm_body(x_ref, w_ref, c_ref, y_ref, pq_ref):
    y = jnp.dot(x_ref[...], w_ref[...], preferred_element_type=jnp.float32)
    y_ref[...] = y
    pq_ref[...] = jnp.dot(y, c_ref[...], preferred_element_type=jnp.float32)


def _mm_proj(x, w, c_pad):
    n, d_in = x.shape
    d_out = w.shape[1]
    return pl.pallas_call(
        _mm_body,
        grid=(n // _BLK,),
        in_specs=[
            pl.BlockSpec((_BLK, d_in), lambda i: (i, 0)),
            pl.BlockSpec((d_in, d_out), lambda i: (0, 0)),
            pl.BlockSpec((d_out, 128), lambda i: (0, 0)),
        ],
        out_specs=[
            pl.BlockSpec((_BLK, d_out), lambda i: (i, 0)),
            pl.BlockSpec((_BLK, 128), lambda i: (i, 0)),
        ],
        out_shape=[
            jax.ShapeDtypeStruct((n, d_out), jnp.float32),
            jax.ShapeDtypeStruct((n, 128), jnp.float32),
        ],
    )(x, w, c_pad)


@functools.partial(
    pl.kernel,
    out_type=[jax.ShapeDtypeStruct((2, NPASS, NH, 128), jnp.float32),
              jax.ShapeDtypeStruct((2, NPASS, NH, 128), jnp.float32)],
    mesh=_mesh,
    scratch_types=[
        pltpu.VMEM((ET_I,), jnp.int32),
        pltpu.VMEM((ET_I,), jnp.int32),
        pltpu.VMEM((ET_I,), jnp.float32),
        pltpu.VMEM((NP,), jnp.float32),
        pltpu.VMEM((NP,), jnp.float32),
        pltpu.VMEM((NPR, 128), jnp.float32),
        pltpu.VMEM((CAP,), jnp.int32),
        pltpu.VMEM((CAP,), jnp.int32),
        pltpu.VMEM((CAP,), jnp.float32),
        pltpu.VMEM((64, 128), jnp.float32),
        pltpu.VMEM((64, 128), jnp.float32),
        pltpu.VMEM((32,), jnp.int32),
        pltpu.VMEM((32,), jnp.int32),
        pltpu.VMEM((32,), jnp.int32),
        pltpu.VMEM((32,), jnp.int32),
        pltpu.VMEM((16,), jnp.int32),
        pltpu.VMEM((16,), jnp.int32),
        pltpu.VMEM((NPR,), jnp.int32),
        pltpu.VMEM_SHARED((NPR, 128), jnp.float32),
        pltpu.VMEM_SHARED((NHA, 128), jnp.float32),
        pltpu.SemaphoreType.DMA,
        pltpu.SemaphoreType.DMA,
        pltpu.SemaphoreType.DMA,
        pltpu.SemaphoreType.DMA,
    ],
    compiler_params=_CP,
)
def _sparse_sc(msgT, smsgT, tmsgT, scal, ai_h, aj_h, ti_h, sj_h,
               out0, out2,
               r_v, c_v, att_v, pr_v, pc_v, sp_v, srid, scid, satt,
               rowbuf4, rowbuf4b,
               rid32a, rid32b, gid32a, gid32b, ridbt, gidb,
               rowids, s_sh, acc, semA, semB, semC, semD):
    zbuf = rowbuf4b.at[pl.ds(0, 16)]
    stg = rowbuf4b.at[pl.ds(16, 16)]
    stg2 = rowbuf4b.at[pl.ds(32, 16)]
    c = lax.axis_index("c")
    t = lax.axis_index("s")
    zero16 = jnp.zeros((16,), jnp.float32)
    iota16 = lax.iota(jnp.int32, 16)
    nsr = NPR // 16

    def _fill_zeros():
        def _zb(i, _):
            for v in range(8):
                rowbuf4b[i, pl.ds(16 * v, 16)] = zero16
            return 0
        lax.fori_loop(0, 16, _zb, 0)

    _fill_zeros()

    def _fri(k, _):
        rowids[pl.ds(16 * k, 16)] = iota16 + 16 * k
        return 0
    lax.fori_loop(0, NPR // 16, _fri, 0)

    def _za(j, _):
        pltpu.sync_copy(zbuf, acc.at[pl.ds(t * RPT + 16 * j, 16)])
        return 0
    lax.fori_loop(0, RPT // 16, _za, 0)

    @pl.when(t == 0)
    def _():
        pltpu.sync_copy(zbuf, acc.at[pl.ds(NH, 16)])
        pltpu.sync_copy(zbuf, acc.at[pl.ds(NH + 16, 16)])

    pltpu.sync_copy(rowbuf4b.at[pl.ds(0, nsr)], s_sh.at[pl.ds(nsr * t, nsr)])
    plsc.subcore_barrier()

    def scalar_stage(et, pr_row, pc_row):
        nch = et // 16
        pltpu.sync_copy(scal.at[pr_row], pr_v)
        pltpu.sync_copy(scal.at[pc_row], pc_v)

        def _zs(i, _):
            for v in range(8):
                sp_v[i, pl.ds(16 * v, 16)] = zero16
            return 0
        lax.fori_loop(0, NPR, _zs, 0)

        def _sta(k, _):
            rid = r_v[pl.ds(16 * k, 16)]
            cid = c_v[pl.ds(16 * k, 16)]
            e = (plsc.load_gather(pr_v, [rid])
                 + plsc.load_gather(pc_v, [cid]))
            e = jnp.where(e >= 0, e, NEG_SLOPE * e)
            ev = jnp.exp(e)
            att_v[pl.ds(16 * k, 16)] = ev
            plsc.addupdate_scatter(
                sp_v, [jnp.right_shift(rid, 7), jnp.bitwise_and(rid, 127)], ev)
            return 0
        lax.fori_loop(0, nch, _sta, 0)

        pltpu.sync_copy(sp_v, s_sh.at[rowids], add=True)
        plsc.subcore_barrier()
        pltpu.sync_copy(s_sh, sp_v)

        def _stb(k, _):
            rid = r_v[pl.ds(16 * k, 16)]
            sv = plsc.load_gather(
                sp_v, [jnp.right_shift(rid, 7), jnp.bitwise_and(rid, 127)])
            att = att_v[pl.ds(16 * k, 16)] / jnp.maximum(sv, 1e-30)
            att_v[pl.ds(16 * k, 16)] = att
            return 0
        lax.fori_loop(0, nch, _stb, 0)

        _fill_zeros()
        pltpu.sync_copy(rowbuf4b.at[pl.ds(0, nsr)],
                        s_sh.at[pl.ds(nsr * t, nsr)])
        plsc.subcore_barrier()

    def _drain_chunk(tab, k, _):
        rid = srid[pl.ds(16 * k, 16)]
        cid = scid[pl.ds(16 * k, 16)]
        ridbt[...] = rid
        gidb[...] = cid * 2 + c
        pltpu.async_copy(tab.at[gidb], rowbuf4.at[pl.ds(0, 16)], semA).wait()
        att_vec = satt[pl.ds(16 * k, 16)]
        for j in range(16):
            av = jnp.full((16,), att_vec[j], jnp.float32)
            for v in range(8):
                sl = pl.ds(16 * v, 16)
                rowbuf4[j, sl] = rowbuf4[j, sl] * av
        pltpu.sync_copy(rowbuf4.at[pl.ds(0, 16)], acc.at[ridbt], add=True)
        return 0

    def heavy(tab, et, p):
        nch = et // 16
        lo = p * NH

        def _scan(k, cur):
            rid = r_v[pl.ds(16 * k, 16)] - lo
            cid = c_v[pl.ds(16 * k, 16)]
            att = att_v[pl.ds(16 * k, 16)]
            mask = jnp.logical_and(rid >= 0, rid < NH)
            plsc.store_compressed(srid.at[pl.ds(cur, 16)], rid, mask=mask)
            plsc.store_compressed(scid.at[pl.ds(cur, 16)], cid, mask=mask)
            plsc.store_compressed(satt.at[pl.ds(cur, 16)], att, mask=mask)
            return cur + plsc.all_reduce_population_count(mask)[0]

        gsl = (pl.ds(0, 32), pl.ds(32, 32))
        gids = (gid32a, gid32b)
        rids = (rid32a, rid32b)
        gsems = (semA, semB)
        ssems = (semC, semD)

        def _prep(g, par):
            base = 32 * g
            for s in range(2):
                gids[par][pl.ds(16 * s, 16)] = (
                    scid[pl.ds(base + 16 * s, 16)] * 2 + c)
            pltpu.async_copy(tab.at[gids[par]], rowbuf4.at[gsl[par]],
                             gsems[par])

        def _wait_gather(par):
            pltpu.make_async_copy(msgT.at[pl.ds(0, 32)],
                                  rowbuf4.at[gsl[par]], gsems[par]).wait()

        def _wait_scatter(par):
            pltpu.make_async_copy(msgT.at[pl.ds(0, 32)],
                                  rowbuf4b.at[gsl[par]], ssems[par]).wait()

        def _scale(g, par):
            off = 32 * par
            for s in range(2):
                att_vec = satt[pl.ds(32 * g + 16 * s, 16)]
                for j in range(16):
                    r = off + 16 * s + j
                    av = jnp.full((16,), att_vec[j], jnp.float32)
                    for v in range(8):
                        sl = pl.ds(16 * v, 16)
                        rowbuf4b[r, sl] = rowbuf4[r, sl] * av

        def _scatter(g, par):
            base = 32 * g
            for s in range(2):
                rids[par][pl.ds(16 * s, 16)] = srid[pl.ds(base + 16 * s, 16)]
            pltpu.make_async_copy(rowbuf4b.at[gsl[par]], acc.at[rids[par]],
                                  ssems[par]).start(add=True)

        def _slot(g, par, ngrp):
            _wait_gather(par)

            @pl.when(g >= 2)
            def _():
                _wait_scatter(par)
            _scale(g, par)

            @pl.when(g + 2 < ngrp)
            def _():
                _prep(g + 2, par)
            _scatter(g, par)

        def _drain(cur):
            ngrp = cur // 32

            @pl.when(ngrp > 0)
            def _():
                _prep(0, 0)

            @pl.when(ngrp > 1)
            def _():
                _prep(1, 1)

            def _pair(ii, _):
                i0 = 2 * ii
                i1 = i0 + 1
                _slot(i0, 0, ngrp)

                @pl.when(i1 < ngrp)
                def _():
                    _slot(i1, 1, ngrp)
                return 0
            lax.fori_loop(0, (ngrp + 1) // 2, _pair, 0)

            @pl.when(ngrp >= 1)
            def _():
                _wait_scatter(0)

            @pl.when(ngrp >= 2)
            def _():
                _wait_scatter(1)
            base = 32 * ngrp
            for s in range(2):
                rv = srid[pl.ds(base + 16 * s, 16)]
                cv = scid[pl.ds(base + 16 * s, 16)]
                av = satt[pl.ds(base + 16 * s, 16)]
                srid[pl.ds(16 * s, 16)] = rv
                scid[pl.ds(16 * s, 16)] = cv
                satt[pl.ds(16 * s, 16)] = av
            return cur - base

        nblk = (nch + SCB - 1) // SCB

        def _blk(b, cur):
            start = b * SCB
            end = jnp.minimum(start + SCB, nch)
            cur = lax.fori_loop(start, end, _scan, cur)
            return _drain(cur)

        cur = lax.fori_loop(0, nblk, _blk, jnp.int32(0))
        fullt = cur // 16
        lax.fori_loop(0, fullt, functools.partial(_drain_chunk, tab), 0)
        rem = cur - fullt * 16
        rv = srid[pl.ds(16 * fullt, 16)]
        cv = scid[pl.ds(16 * fullt, 16)]
        av = satt[pl.ds(16 * fullt, 16)]
        live = iota16 < rem
        srid[pl.ds(0, 16)] = jnp.where(live, rv, NH)
        scid[pl.ds(0, 16)] = jnp.where(live, cv, 0)
        satt[pl.ds(0, 16)] = jnp.where(live, av, 0.0)
        _drain_chunk(tab, 0, 0)

    def writeback_zero(out, p, add_prev):
        _fill_zeros()

        def _wb(j, _):
            r0 = t * RPT + 16 * j
            pltpu.sync_copy(acc.at[pl.ds(r0, 16)], stg)
            if add_prev:
                pltpu.sync_copy(out.at[c, p, pl.ds(r0, 16)], stg2)

                def _addrow(i, _):
                    for v in range(8):
                        sl = pl.ds(16 * v, 16)
                        rowbuf4b[16 + i, sl] = (rowbuf4b[16 + i, sl]
                                                + rowbuf4b[32 + i, sl])
                    return 0
                lax.fori_loop(0, 16, _addrow, 0)
            pltpu.sync_copy(stg, out.at[c, p, pl.ds(r0, 16)])
            pltpu.sync_copy(zbuf, acc.at[pl.ds(r0, 16)])
            return 0
        lax.fori_loop(0, RPT // 16, _wb, 0)

    def round_(tab, rhbm, chbm, et, pr_row, pc_row, out, add_prev):
        pltpu.sync_copy(rhbm.at[pl.ds(t * et, et)], r_v.at[pl.ds(0, et)])
        pltpu.sync_copy(chbm.at[pl.ds(t * et, et)], c_v.at[pl.ds(0, et)])
        scalar_stage(et, pr_row, pc_row)

        def _pass(p, _):
            heavy(tab, et, p)
            plsc.subcore_barrier()
            writeback_zero(out, p, add_prev)
            plsc.subcore_barrier()
            return 0
        lax.fori_loop(0, NPASS, _pass, 0)

    round_(msgT, ai_h, aj_h, ET_A, 0, 1, out0, False)
    round_(smsgT, ti_h, sj_h, ET_I, 2, 3, out0, True)
    round_(tmsgT, sj_h, ti_h, ET_I, 5, 4, out2, False)


def kernel(x_0, x_2, adjacency_0, incidence_0_2, w_hbs, att_hbs, w_s, w_t, att_hbns):
    def cpad(c0, c1):
        z = jnp.zeros((256, 128), jnp.float32)
        z = z.at[:, 0].set(c0)
        return z.at[:, 1].set(c1)

    c_hbs = cpad(att_hbs[:256, 0], att_hbs[256:, 0])
    c_t = cpad(att_hbns[256:, 0], att_hbns[:256, 0])
    c_s = cpad(att_hbns[:256, 0], att_hbns[256:, 0])

    msg, pq = _mm_proj(x_0, w_hbs, c_hbs)
    t_msg, atbt = _mm_proj(x_0, w_t, c_t)
    s_msg, asbs = _mm_proj(x_2, w_s, c_s)

    npad = NP - N0

    def tabify(y):
        return jnp.pad(y, ((0, npad), (0, 0))).reshape(2 * NP, 128)

    def svec(v):
        return jnp.pad(v, (0, npad))

    scal = jnp.stack([svec(pq[:, 0]), svec(pq[:, 1]),
                      svec(atbt[:, 0]), svec(asbs[:, 0]),
                      svec(atbt[:, 1]), svec(asbs[:, 1])])

    ipad = NNZP - NNZ
    ti = jnp.pad(incidence_0_2[0], (0, ipad), constant_values=N0)
    sj = jnp.pad(incidence_0_2[1], (0, ipad), constant_values=N0)

    out0, out2 = _sparse_sc(tabify(msg), tabify(s_msg), tabify(t_msg), scal,
                            adjacency_0[0], adjacency_0[1], ti, sj)

    def assemble(o, n):
        cols = [o[cc].reshape(NPASS * NH, 128) for cc in (0, 1)]
        return jnp.concatenate(cols, axis=1)[:n]

    return (assemble(out0, N0), assemble(out2, N2))
